# baseline (speedup 1.0000x reference)
import jax
import jax.numpy as jnp
from jax import lax
from jax.experimental import pallas as pl
from jax.experimental.pallas import tpu as pltpu
from jax.experimental.pallas import tpu_sc as plsc

N = 10000
E = 160000
H = 256
HH = H // 2
EPS = 1e-5
NC, NS, L = 2, 16, 16
K = 80
EPT = E // NS
NB = EPT // K
NPAD = 10240
RPT = NPAD // NS
FC = 32
RB_N = 2000
RB_E = 2000


def _node_mm_body(x_ref, uw_ref, ub_ref, aw_ref, ab_ref, bw_ref, bb_ref,
                  vw_ref, vb_ref, ux_ref, an_ref, bv_ref):
    xb = x_ref[...]
    dn = (((1,), (1,)), ((), ()))
    ux = lax.dot_general(xb, uw_ref[...], dn,
                         preferred_element_type=jnp.float32) + ub_ref[...]
    ya = lax.dot_general(xb, aw_ref[...], dn,
                         preferred_element_type=jnp.float32) + ab_ref[...]
    yb = lax.dot_general(xb, bw_ref[...], dn,
                         preferred_element_type=jnp.float32) + bb_ref[...]
    yv = lax.dot_general(xb, vw_ref[...], dn,
                         preferred_element_type=jnp.float32) + vb_ref[...]
    ux_ref[...] = ux
    an_ref[0] = ya[:, :HH]
    an_ref[1] = ya[:, HH:]
    bv_ref[0, :, :HH] = yb[:, :HH]
    bv_ref[0, :, HH:] = yv[:, :HH]
    bv_ref[1, :, :HH] = yb[:, HH:]
    bv_ref[1, :, HH:] = yv[:, HH:]


def _edge_mm_body(e_ref, cw_ref, cb_ref, ce_ref):
    y = lax.dot_general(e_ref[...], cw_ref[...], (((1,), (1,)), ((), ())),
                        preferred_element_type=jnp.float32) + cb_ref[...]
    ce_ref[0] = y[:, :HH]
    ce_ref[1] = y[:, HH:]


def _sc_edge_body(src_ref, dst_ref, an_ref, bv_ref, ce_ref,
                  enew_ref, agg_ref, stats_ref,
                  isrc, isrcA, idstA, a_v, bv_v, ce_v, st_v, zb,
                  acc, sem):
    c = lax.axis_index("c")
    s = lax.axis_index("s")
    coff = c * N
    zero = jnp.zeros((L,), jnp.float32)

    def _zrow(r, carry):
        for j in range(HH // L):
            zb[r, pl.ds(j * L, L)] = zero
        return carry
    lax.fori_loop(0, FC, _zrow, 0)
    for t in range(RPT // FC):
        pltpu.sync_copy(zb, acc.at[pl.ds(s * RPT + t * FC, FC)])
    plsc.subcore_barrier()

    def _batch(i, st):
        base = s * EPT + i * K
        pltpu.sync_copy(src_ref.at[pl.ds(base, K)], isrc)
        pltpu.sync_copy(dst_ref.at[pl.ds(base, K)], idstA)
        for j in range(K // L):
            sl = pl.ds(j * L, L)
            isrcA[sl] = isrc[sl] + coff
            idstA[sl] = idstA[sl] + coff
        cpa = pltpu.async_copy(an_ref.at[isrcA], a_v, sem)
        cpb = pltpu.async_copy(bv_ref.at[idstA], bv_v, sem)
        cpc = pltpu.async_copy(ce_ref.at[pl.ds(c * E + base, K)], ce_v, sem)
        cpa.wait()
        cpb.wait()
        cpc.wait()

        def _row(r, st2):
            out = list(st2)
            for j in range(HH // L):
                sl = pl.ds(j * L, L)
                t = a_v[r, sl] + bv_v[r, sl] + ce_v[r, sl]
                ce_v[r, sl] = t
                g = 1.0 / (1.0 + jnp.exp(-t))
                a_v[r, sl] = bv_v[r, pl.ds(HH + j * L, L)] * g
                out[j] = out[j] + t
                out[8 + j] = out[8 + j] + t * t
            return tuple(out)
        st = lax.fori_loop(0, K, _row, st)

        pltpu.sync_copy(ce_v, enew_ref.at[pl.ds(c * E + base, K)])
        pltpu.sync_copy(a_v, acc.at[isrc], add=True)
        return st

    st0 = tuple(jnp.zeros((L,), jnp.float32) for _ in range(16))
    st = lax.fori_loop(0, NB, _batch, st0)

    for j in range(HH // L):
        st_v[0, pl.ds(j * L, L)] = st[j]
        st_v[1, pl.ds(j * L, L)] = st[8 + j]
    pltpu.sync_copy(st_v, stats_ref.at[c, s])

    plsc.subcore_barrier()
    for t in range(RPT // FC):
        rows = pl.ds(s * RPT + t * FC, FC)
        pltpu.sync_copy(acc.at[rows], zb)
        pltpu.sync_copy(zb, agg_ref.at[pl.ds(c * NPAD + s * RPT + t * FC, FC)])


def _make_sc_edge():
  return pl.kernel(
    _sc_edge_body,
    out_type=(
        jax.ShapeDtypeStruct((2 * E, HH), jnp.float32),
        jax.ShapeDtypeStruct((2 * NPAD, HH), jnp.float32),
        jax.ShapeDtypeStruct((NC, NS, 2, HH), jnp.float32),
    ),
    mesh=plsc.VectorSubcoreMesh(core_axis_name="c", subcore_axis_name="s",
                                num_cores=NC, num_subcores=NS),
    scratch_types=[
        pltpu.VMEM((K,), jnp.int32),
        pltpu.VMEM((K,), jnp.int32),
        pltpu.VMEM((K,), jnp.int32),
        pltpu.VMEM((K, HH), jnp.float32),
        pltpu.VMEM((K, H), jnp.float32),
        pltpu.VMEM((K, HH), jnp.float32),
        pltpu.VMEM((2, HH), jnp.float32),
        pltpu.VMEM((FC, HH), jnp.float32),
        pltpu.VMEM_SHARED((NPAD, HH), jnp.float32),
        pltpu.SemaphoreType.DMA,
    ],
  )


def _xside_body(ux_ref, agg_ref, gx_ref, bx_ref, xout_ref):
    xl = ux_ref[:, :HH] + agg_ref[0, :N]
    xr = ux_ref[:, HH:] + agg_ref[1, :N]
    xn = jnp.concatenate([xl, xr], axis=1)
    mu = jnp.mean(xn, axis=0, keepdims=True)
    var = jnp.mean((xn - mu) ** 2, axis=0, keepdims=True)
    y = (xn - mu) * lax.rsqrt(var + EPS) * gx_ref[...] + bx_ref[...]
    xout_ref[...] = jnp.maximum(y, 0.0)


def _eside_body(stats_ref, ge_ref, be_ref, en_ref, eout_ref):
    stt = stats_ref[...]
    s1 = jnp.sum(stt[:, :, 0, :], axis=1)
    s2 = jnp.sum(stt[:, :, 1, :], axis=1)
    mu = s1 / E
    var = s2 / E - mu * mu
    scale = ge_ref[...] * lax.rsqrt(var + EPS)
    shift = be_ref[...] - mu * scale
    e0 = en_ref[0] * scale[0] + shift[0]
    e1 = en_ref[1] * scale[1] + shift[1]
    eout_ref[:, :HH] = jnp.maximum(e0, 0.0)
    eout_ref[:, HH:] = jnp.maximum(e1, 0.0)


def kernel(x, e, edge_index, U_w, U_b, V_w, V_b, A_w, A_b, B_w, B_b,
           C_w, C_b, gamma_x, beta_x, gamma_e, beta_e):
    src = edge_index[0].astype(jnp.int32)
    dst = edge_index[1].astype(jnp.int32)

    cw = pl.BlockSpec((H, H), lambda i: (0, 0))
    cb = pl.BlockSpec((1, H), lambda i: (0, 0))

    ux, an3, bv3 = pl.pallas_call(
        _node_mm_body,
        grid=(N // RB_N,),
        in_specs=[pl.BlockSpec((RB_N, H), lambda i: (i, 0)),
                  cw, cb, cw, cb, cw, cb, cw, cb],
        out_specs=[pl.BlockSpec((RB_N, H), lambda i: (i, 0)),
                   pl.BlockSpec((2, RB_N, HH), lambda i: (0, i, 0)),
                   pl.BlockSpec((2, RB_N, H), lambda i: (0, i, 0))],
        out_shape=[jax.ShapeDtypeStruct((N, H), jnp.float32),
                   jax.ShapeDtypeStruct((2, N, HH), jnp.float32),
                   jax.ShapeDtypeStruct((2, N, H), jnp.float32)],
    )(x, U_w, U_b.reshape(1, H), A_w, A_b.reshape(1, H),
      B_w, B_b.reshape(1, H), V_w, V_b.reshape(1, H))

    ce3 = pl.pallas_call(
        _edge_mm_body,
        grid=(E // RB_E,),
        in_specs=[pl.BlockSpec((RB_E, H), lambda i: (i, 0)), cw, cb],
        out_specs=pl.BlockSpec((2, RB_E, HH), lambda i: (0, i, 0)),
        out_shape=jax.ShapeDtypeStruct((2, E, HH), jnp.float32),
    )(e, C_w, C_b.reshape(1, H))

    enew, agg, stats = _make_sc_edge()(src, dst,
                                an3.reshape(2 * N, HH),
                                bv3.reshape(2 * N, H),
                                ce3.reshape(2 * E, HH))

    xout = pl.pallas_call(
        _xside_body,
        out_shape=jax.ShapeDtypeStruct((N, H), jnp.float32),
    )(ux, agg.reshape(2, NPAD, HH), gamma_x.reshape(1, H), beta_x.reshape(1, H))

    eout = pl.pallas_call(
        _eside_body,
        grid=(E // RB_E,),
        in_specs=[pl.BlockSpec((NC, NS, 2, HH), lambda i: (0, 0, 0, 0)),
                  pl.BlockSpec((2, HH), lambda i: (0, 0)),
                  pl.BlockSpec((2, HH), lambda i: (0, 0)),
                  pl.BlockSpec((2, RB_E, HH), lambda i: (0, i, 0))],
        out_specs=pl.BlockSpec((RB_E, H), lambda i: (i, 0)),
        out_shape=jax.ShapeDtypeStruct((E, H), jnp.float32),
    )(stats, gamma_e.reshape(2, HH), beta_e.reshape(2, HH),
      enew.reshape(2, E, HH))

    return xout, eout


# speedup vs baseline: 1.0003x; 1.0003x over previous
import jax
import jax.numpy as jnp
from jax import lax
from jax.experimental import pallas as pl
from jax.experimental.pallas import tpu as pltpu
from jax.experimental.pallas import tpu_sc as plsc

N = 10000
E = 160000
H = 256
HH = H // 2
EPS = 1e-5
NC, NS, L = 2, 16, 16
K = 80
EPT = E // NS
NB = EPT // K
NPAD = 10240
RPT = NPAD // NS
FC = 32
RB_N = 2000
RB_E = 2000


def _node_mm_body(x_ref, uw_ref, ub_ref, aw_ref, ab_ref, bw_ref, bb_ref,
                  vw_ref, vb_ref, ux_ref, an_ref, bv_ref):
    xb = x_ref[...].astype(jnp.bfloat16)
    dn = (((1,), (1,)), ((), ()))
    ux = lax.dot_general(xb, uw_ref[...].astype(jnp.bfloat16), dn,
                         preferred_element_type=jnp.float32) + ub_ref[...]
    ya = lax.dot_general(xb, aw_ref[...].astype(jnp.bfloat16), dn,
                         preferred_element_type=jnp.float32) + ab_ref[...]
    yb = lax.dot_general(xb, bw_ref[...].astype(jnp.bfloat16), dn,
                         preferred_element_type=jnp.float32) + bb_ref[...]
    yv = lax.dot_general(xb, vw_ref[...].astype(jnp.bfloat16), dn,
                         preferred_element_type=jnp.float32) + vb_ref[...]
    ux_ref[...] = ux
    an_ref[0] = ya[:, :HH]
    an_ref[1] = ya[:, HH:]
    bv_ref[0, :, :HH] = yb[:, :HH]
    bv_ref[0, :, HH:] = yv[:, :HH]
    bv_ref[1, :, :HH] = yb[:, HH:]
    bv_ref[1, :, HH:] = yv[:, HH:]


def _edge_mm_body(e_ref, cw_ref, cb_ref, ce_ref):
    y = lax.dot_general(e_ref[...].astype(jnp.bfloat16),
                        cw_ref[...].astype(jnp.bfloat16),
                        (((1,), (1,)), ((), ())),
                        preferred_element_type=jnp.float32) + cb_ref[...]
    ce_ref[0] = y[:, :HH]
    ce_ref[1] = y[:, HH:]


def _sc_edge_body(src_ref, dst_ref, an_ref, bv_ref, ce_ref,
                  enew_ref, agg_ref, stats_ref,
                  isrc, isrcA, idstA, a_v, bv_v, ce_v, st_v, zb,
                  acc, sem):
    c = lax.axis_index("c")
    s = lax.axis_index("s")
    coff = c * N
    zero = jnp.zeros((L,), jnp.float32)

    def _zrow(r, carry):
        for j in range(HH // L):
            zb[r, pl.ds(j * L, L)] = zero
        return carry
    lax.fori_loop(0, FC, _zrow, 0)
    for t in range(RPT // FC):
        pltpu.sync_copy(zb, acc.at[pl.ds(s * RPT + t * FC, FC)])
    plsc.subcore_barrier()

    def _batch(i, st):
        base = s * EPT + i * K
        pltpu.sync_copy(src_ref.at[pl.ds(base, K)], isrc)
        pltpu.sync_copy(dst_ref.at[pl.ds(base, K)], idstA)
        for j in range(K // L):
            sl = pl.ds(j * L, L)
            isrcA[sl] = isrc[sl] + coff
            idstA[sl] = idstA[sl] + coff
        cpa = pltpu.async_copy(an_ref.at[isrcA], a_v, sem)
        cpb = pltpu.async_copy(bv_ref.at[idstA], bv_v, sem)
        cpc = pltpu.async_copy(ce_ref.at[pl.ds(c * E + base, K)], ce_v, sem)
        cpa.wait()
        cpb.wait()
        cpc.wait()

        def _row(r, st2):
            out = list(st2)
            for j in range(HH // L):
                sl = pl.ds(j * L, L)
                t = a_v[r, sl] + bv_v[r, sl] + ce_v[r, sl]
                ce_v[r, sl] = t
                g = 1.0 / (1.0 + jnp.exp(-t))
                a_v[r, sl] = bv_v[r, pl.ds(HH + j * L, L)] * g
                out[j] = out[j] + t
                out[8 + j] = out[8 + j] + t * t
            return tuple(out)
        st = plsc.parallel_loop(0, K, carry=tuple(st))(_row)

        pltpu.sync_copy(ce_v, enew_ref.at[pl.ds(c * E + base, K)])
        pltpu.sync_copy(a_v, acc.at[isrc], add=True)
        return st

    st0 = tuple(jnp.zeros((L,), jnp.float32) for _ in range(16))
    st = lax.fori_loop(0, NB, _batch, st0)

    for j in range(HH // L):
        st_v[0, pl.ds(j * L, L)] = st[j]
        st_v[1, pl.ds(j * L, L)] = st[8 + j]
    pltpu.sync_copy(st_v, stats_ref.at[c, s])

    plsc.subcore_barrier()
    for t in range(RPT // FC):
        rows = pl.ds(s * RPT + t * FC, FC)
        pltpu.sync_copy(acc.at[rows], zb)
        pltpu.sync_copy(zb, agg_ref.at[pl.ds(c * NPAD + s * RPT + t * FC, FC)])


def _make_sc_edge():
  return pl.kernel(
    _sc_edge_body,
    out_type=(
        jax.ShapeDtypeStruct((2 * E, HH), jnp.float32),
        jax.ShapeDtypeStruct((2 * NPAD, HH), jnp.float32),
        jax.ShapeDtypeStruct((NC, NS, 2, HH), jnp.float32),
    ),
    mesh=plsc.VectorSubcoreMesh(core_axis_name="c", subcore_axis_name="s",
                                num_cores=NC, num_subcores=NS),
    scratch_types=[
        pltpu.VMEM((K,), jnp.int32),
        pltpu.VMEM((K,), jnp.int32),
        pltpu.VMEM((K,), jnp.int32),
        pltpu.VMEM((K, HH), jnp.float32),
        pltpu.VMEM((K, H), jnp.float32),
        pltpu.VMEM((K, HH), jnp.float32),
        pltpu.VMEM((2, HH), jnp.float32),
        pltpu.VMEM((FC, HH), jnp.float32),
        pltpu.VMEM_SHARED((NPAD, HH), jnp.float32),
        pltpu.SemaphoreType.DMA,
    ],
  )


def _xside_body(ux_ref, agg_ref, gx_ref, bx_ref, xout_ref):
    xl = ux_ref[:, :HH] + agg_ref[0, :N]
    xr = ux_ref[:, HH:] + agg_ref[1, :N]
    xn = jnp.concatenate([xl, xr], axis=1)
    mu = jnp.mean(xn, axis=0, keepdims=True)
    var = jnp.mean((xn - mu) ** 2, axis=0, keepdims=True)
    y = (xn - mu) * lax.rsqrt(var + EPS) * gx_ref[...] + bx_ref[...]
    xout_ref[...] = jnp.maximum(y, 0.0)


def _eside_body(stats_ref, ge_ref, be_ref, en_ref, eout_ref):
    stt = stats_ref[...]
    s1 = jnp.sum(stt[:, :, 0, :], axis=1)
    s2 = jnp.sum(stt[:, :, 1, :], axis=1)
    mu = s1 / E
    var = s2 / E - mu * mu
    scale = ge_ref[...] * lax.rsqrt(var + EPS)
    shift = be_ref[...] - mu * scale
    e0 = en_ref[0] * scale[0] + shift[0]
    e1 = en_ref[1] * scale[1] + shift[1]
    eout_ref[:, :HH] = jnp.maximum(e0, 0.0)
    eout_ref[:, HH:] = jnp.maximum(e1, 0.0)


def kernel(x, e, edge_index, U_w, U_b, V_w, V_b, A_w, A_b, B_w, B_b,
           C_w, C_b, gamma_x, beta_x, gamma_e, beta_e):
    src = edge_index[0].astype(jnp.int32)
    dst = edge_index[1].astype(jnp.int32)

    cw = pl.BlockSpec((H, H), lambda i: (0, 0))
    cb = pl.BlockSpec((1, H), lambda i: (0, 0))

    ux, an3, bv3 = pl.pallas_call(
        _node_mm_body,
        grid=(N // RB_N,),
        in_specs=[pl.BlockSpec((RB_N, H), lambda i: (i, 0)),
                  cw, cb, cw, cb, cw, cb, cw, cb],
        out_specs=[pl.BlockSpec((RB_N, H), lambda i: (i, 0)),
                   pl.BlockSpec((2, RB_N, HH), lambda i: (0, i, 0)),
                   pl.BlockSpec((2, RB_N, H), lambda i: (0, i, 0))],
        out_shape=[jax.ShapeDtypeStruct((N, H), jnp.float32),
                   jax.ShapeDtypeStruct((2, N, HH), jnp.float32),
                   jax.ShapeDtypeStruct((2, N, H), jnp.float32)],
    )(x, U_w, U_b.reshape(1, H), A_w, A_b.reshape(1, H),
      B_w, B_b.reshape(1, H), V_w, V_b.reshape(1, H))

    ce3 = pl.pallas_call(
        _edge_mm_body,
        grid=(E // RB_E,),
        in_specs=[pl.BlockSpec((RB_E, H), lambda i: (i, 0)), cw, cb],
        out_specs=pl.BlockSpec((2, RB_E, HH), lambda i: (0, i, 0)),
        out_shape=jax.ShapeDtypeStruct((2, E, HH), jnp.float32),
    )(e, C_w, C_b.reshape(1, H))

    enew, agg, stats = _make_sc_edge()(src, dst,
                                an3.reshape(2 * N, HH),
                                bv3.reshape(2 * N, H),
                                ce3.reshape(2 * E, HH))

    xout = pl.pallas_call(
        _xside_body,
        out_shape=jax.ShapeDtypeStruct((N, H), jnp.float32),
    )(ux, agg.reshape(2, NPAD, HH), gamma_x.reshape(1, H), beta_x.reshape(1, H))

    eout = pl.pallas_call(
        _eside_body,
        grid=(E // RB_E,),
        in_specs=[pl.BlockSpec((NC, NS, 2, HH), lambda i: (0, 0, 0, 0)),
                  pl.BlockSpec((2, HH), lambda i: (0, 0)),
                  pl.BlockSpec((2, HH), lambda i: (0, 0)),
                  pl.BlockSpec((2, RB_E, HH), lambda i: (0, i, 0))],
        out_specs=pl.BlockSpec((RB_E, H), lambda i: (i, 0)),
        out_shape=jax.ShapeDtypeStruct((E, H), jnp.float32),
    )(stats, gamma_e.reshape(2, HH), beta_e.reshape(2, HH),
      enew.reshape(2, E, HH))

    return xout, eout


# speedup vs baseline: 1.0369x; 1.0366x over previous
import jax
import jax.numpy as jnp
from jax import lax
from jax.experimental import pallas as pl
from jax.experimental.pallas import tpu as pltpu
from jax.experimental.pallas import tpu_sc as plsc

N = 10000
E = 160000
H = 256
HH = H // 2
EPS = 1e-5
NC, NS, L = 2, 16, 16
K = 80
EPT = E // NS
NB = EPT // K
NPAD = 10240
RPT = NPAD // NS
FC = 16
RB_N = 2000
RB_E = 2000


def _node_mm_body(x_ref, uw_ref, ub_ref, aw_ref, ab_ref, bw_ref, bb_ref,
                  vw_ref, vb_ref, ux_ref, an_ref, bv_ref):
    xb = x_ref[...].astype(jnp.bfloat16)
    dn = (((1,), (1,)), ((), ()))
    ux = lax.dot_general(xb, uw_ref[...].astype(jnp.bfloat16), dn,
                         preferred_element_type=jnp.float32) + ub_ref[...]
    ya = lax.dot_general(xb, aw_ref[...].astype(jnp.bfloat16), dn,
                         preferred_element_type=jnp.float32) + ab_ref[...]
    yb = lax.dot_general(xb, bw_ref[...].astype(jnp.bfloat16), dn,
                         preferred_element_type=jnp.float32) + bb_ref[...]
    yv = lax.dot_general(xb, vw_ref[...].astype(jnp.bfloat16), dn,
                         preferred_element_type=jnp.float32) + vb_ref[...]
    ux_ref[...] = ux
    an_ref[0] = ya[:, :HH]
    an_ref[1] = ya[:, HH:]
    bv_ref[0, :, :HH] = yb[:, :HH]
    bv_ref[0, :, HH:] = yv[:, :HH]
    bv_ref[1, :, :HH] = yb[:, HH:]
    bv_ref[1, :, HH:] = yv[:, HH:]


def _edge_mm_body(e_ref, cw_ref, cb_ref, ce_ref):
    y = lax.dot_general(e_ref[...].astype(jnp.bfloat16),
                        cw_ref[...].astype(jnp.bfloat16),
                        (((1,), (1,)), ((), ())),
                        preferred_element_type=jnp.float32) + cb_ref[...]
    ce_ref[0] = y[:, :HH]
    ce_ref[1] = y[:, HH:]


def _sc_edge_body(src_ref, dst_ref, an_ref, bv_ref, ce_ref,
                  enew_ref, agg_ref, stats_ref,
                  blk_src, blk_dst, isc, isrcA, idstA, a_v, bv_v, ce_v,
                  st_v, zb, acc, sem):
    c = lax.axis_index("c")
    s = lax.axis_index("s")
    coff = c * N
    zero = jnp.zeros((L,), jnp.float32)

    def _zrow(r, carry):
        for j in range(HH // L):
            zb[r, pl.ds(j * L, L)] = zero
        return carry
    lax.fori_loop(0, FC, _zrow, 0)
    for t in range(RPT // FC):
        pltpu.sync_copy(zb, acc.at[pl.ds(s * RPT + t * FC, FC)])
    plsc.subcore_barrier()

    IBB = 25 * K

    def _batch(i, st):
        base = s * EPT + i * K
        b = lax.rem(i, 25)

        @pl.when(b == 0)
        def _load_blk():
            pltpu.sync_copy(src_ref.at[pl.ds(base, IBB)], blk_src)
            pltpu.sync_copy(dst_ref.at[pl.ds(base, IBB)], blk_dst)
        for j in range(K // L):
            sl = pl.ds(j * L, L)
            boff = b * K + j * L
            raw = blk_src[pl.ds(boff, L)]
            isc[sl] = raw
            isrcA[sl] = raw + coff
            idstA[sl] = blk_dst[pl.ds(boff, L)] + coff
        cpa = pltpu.async_copy(an_ref.at[isrcA], a_v, sem)
        cpb = pltpu.async_copy(bv_ref.at[idstA], bv_v, sem)
        cpc = pltpu.async_copy(ce_ref.at[pl.ds(c * E + base, K)], ce_v, sem)
        cpa.wait()
        cpb.wait()
        cpc.wait()

        def _row(r, st2):
            out = list(st2)
            for j in range(HH // L):
                sl = pl.ds(j * L, L)
                t = a_v[r, sl] + bv_v[r, sl] + ce_v[r, sl]
                ce_v[r, sl] = t
                g = 1.0 / (1.0 + jnp.exp(-t))
                a_v[r, sl] = bv_v[r, pl.ds(HH + j * L, L)] * g
                out[j] = out[j] + t
                out[8 + j] = out[8 + j] + t * t
            return tuple(out)
        st = plsc.parallel_loop(0, K, carry=tuple(st))(_row)

        pltpu.sync_copy(ce_v, enew_ref.at[pl.ds(c * E + base, K)])
        pltpu.sync_copy(a_v, acc.at[isc], add=True)
        return st

    st0 = tuple(jnp.zeros((L,), jnp.float32) for _ in range(16))
    st = lax.fori_loop(0, NB, _batch, st0)

    for j in range(HH // L):
        st_v[0, pl.ds(j * L, L)] = st[j]
        st_v[1, pl.ds(j * L, L)] = st[8 + j]
    pltpu.sync_copy(st_v, stats_ref.at[c, s])

    plsc.subcore_barrier()
    for t in range(RPT // FC):
        rows = pl.ds(s * RPT + t * FC, FC)
        pltpu.sync_copy(acc.at[rows], zb)
        pltpu.sync_copy(zb, agg_ref.at[pl.ds(c * NPAD + s * RPT + t * FC, FC)])


def _make_sc_edge():
  return pl.kernel(
    _sc_edge_body,
    out_type=(
        jax.ShapeDtypeStruct((2 * E, HH), jnp.float32),
        jax.ShapeDtypeStruct((2 * NPAD, HH), jnp.float32),
        jax.ShapeDtypeStruct((NC, NS, 2, HH), jnp.float32),
    ),
    mesh=plsc.VectorSubcoreMesh(core_axis_name="c", subcore_axis_name="s",
                                num_cores=NC, num_subcores=NS),
    scratch_types=[
        pltpu.VMEM((25 * K,), jnp.int32),
        pltpu.VMEM((25 * K,), jnp.int32),
        pltpu.VMEM((K,), jnp.int32),
        pltpu.VMEM((K,), jnp.int32),
        pltpu.VMEM((K,), jnp.int32),
        pltpu.VMEM((K, HH), jnp.float32),
        pltpu.VMEM((K, H), jnp.float32),
        pltpu.VMEM((K, HH), jnp.float32),
        pltpu.VMEM((2, HH), jnp.float32),
        pltpu.VMEM((FC, HH), jnp.float32),
        pltpu.VMEM_SHARED((NPAD, HH), jnp.float32),
        pltpu.SemaphoreType.DMA,
    ],
  )


def _xside_body(ux_ref, agg_ref, gx_ref, bx_ref, xout_ref):
    xl = ux_ref[:, :HH] + agg_ref[0, :N]
    xr = ux_ref[:, HH:] + agg_ref[1, :N]
    xn = jnp.concatenate([xl, xr], axis=1)
    mu = jnp.mean(xn, axis=0, keepdims=True)
    var = jnp.mean((xn - mu) ** 2, axis=0, keepdims=True)
    y = (xn - mu) * lax.rsqrt(var + EPS) * gx_ref[...] + bx_ref[...]
    xout_ref[...] = jnp.maximum(y, 0.0)


def _eside_body(stats_ref, ge_ref, be_ref, en_ref, eout_ref):
    stt = stats_ref[...]
    s1 = jnp.sum(stt[:, :, 0, :], axis=1)
    s2 = jnp.sum(stt[:, :, 1, :], axis=1)
    mu = s1 / E
    var = s2 / E - mu * mu
    scale = ge_ref[...] * lax.rsqrt(var + EPS)
    shift = be_ref[...] - mu * scale
    e0 = en_ref[0] * scale[0] + shift[0]
    e1 = en_ref[1] * scale[1] + shift[1]
    eout_ref[:, :HH] = jnp.maximum(e0, 0.0)
    eout_ref[:, HH:] = jnp.maximum(e1, 0.0)


def kernel(x, e, edge_index, U_w, U_b, V_w, V_b, A_w, A_b, B_w, B_b,
           C_w, C_b, gamma_x, beta_x, gamma_e, beta_e):
    src = edge_index[0].astype(jnp.int32)
    dst = edge_index[1].astype(jnp.int32)

    cw = pl.BlockSpec((H, H), lambda i: (0, 0))
    cb = pl.BlockSpec((1, H), lambda i: (0, 0))

    ux, an3, bv3 = pl.pallas_call(
        _node_mm_body,
        grid=(N // RB_N,),
        in_specs=[pl.BlockSpec((RB_N, H), lambda i: (i, 0)),
                  cw, cb, cw, cb, cw, cb, cw, cb],
        out_specs=[pl.BlockSpec((RB_N, H), lambda i: (i, 0)),
                   pl.BlockSpec((2, RB_N, HH), lambda i: (0, i, 0)),
                   pl.BlockSpec((2, RB_N, H), lambda i: (0, i, 0))],
        out_shape=[jax.ShapeDtypeStruct((N, H), jnp.float32),
                   jax.ShapeDtypeStruct((2, N, HH), jnp.float32),
                   jax.ShapeDtypeStruct((2, N, H), jnp.float32)],
    )(x, U_w, U_b.reshape(1, H), A_w, A_b.reshape(1, H),
      B_w, B_b.reshape(1, H), V_w, V_b.reshape(1, H))

    ce3 = pl.pallas_call(
        _edge_mm_body,
        grid=(E // RB_E,),
        in_specs=[pl.BlockSpec((RB_E, H), lambda i: (i, 0)), cw, cb],
        out_specs=pl.BlockSpec((2, RB_E, HH), lambda i: (0, i, 0)),
        out_shape=jax.ShapeDtypeStruct((2, E, HH), jnp.float32),
    )(e, C_w, C_b.reshape(1, H))

    enew, agg, stats = _make_sc_edge()(src, dst,
                                an3.reshape(2 * N, HH),
                                bv3.reshape(2 * N, H),
                                ce3.reshape(2 * E, HH))

    xout = pl.pallas_call(
        _xside_body,
        out_shape=jax.ShapeDtypeStruct((N, H), jnp.float32),
    )(ux, agg.reshape(2, NPAD, HH), gamma_x.reshape(1, H), beta_x.reshape(1, H))

    eout = pl.pallas_call(
        _eside_body,
        grid=(E // RB_E,),
        in_specs=[pl.BlockSpec((NC, NS, 2, HH), lambda i: (0, 0, 0, 0)),
                  pl.BlockSpec((2, HH), lambda i: (0, 0)),
                  pl.BlockSpec((2, HH), lambda i: (0, 0)),
                  pl.BlockSpec((2, RB_E, HH), lambda i: (0, i, 0))],
        out_specs=pl.BlockSpec((RB_E, H), lambda i: (i, 0)),
        out_shape=jax.ShapeDtypeStruct((E, H), jnp.float32),
    )(stats, gamma_e.reshape(2, HH), beta_e.reshape(2, HH),
      enew.reshape(2, E, HH))

    return xout, eout
